# baseline (speedup 1.0000x reference)
import math

import jax
import jax.numpy as jnp
from jax import lax
from jax.experimental import pallas as pl
from jax.experimental.pallas import tpu as pltpu
from jax.experimental.pallas import tpu_sc as plsc

V = 100000
D = 64
W = 128
NC, NS, LN = 2, 16, 16
NW = NC * NS
CH = 128
NBUF = 4


_PI_HI = 3.140625
_PI_LO = math.pi - 3.140625
_SINC = (1.0, -1 / 6, 1 / 120, -1 / 5040, 1 / 362880)
_COSC = (1.0, -1 / 2, 1 / 24, -1 / 720, 1 / 40320)


def _ext_table_body(tok_ref, ac_ref, bias_ref, out_ref):
    i = pl.program_id(0)
    r = tok_ref.shape[0]
    row = (lax.broadcasted_iota(jnp.int32, (r, D), 0) + i * r
           ).astype(jnp.float32)
    col = lax.broadcasted_iota(jnp.int32, (r, D), 1)
    k = ((col // 2) * 2).astype(jnp.float32)
    ang = row * jnp.exp(k * (-math.log(10000.0) / D))
    q = jnp.floor(ang * (1.0 / math.pi) + 0.5)
    rr = (ang - q * _PI_HI) - q * _PI_LO
    r2 = rr * rr
    qh = q * 0.5
    sign = 1.0 - 4.0 * (qh - jnp.floor(qh))
    even = col % 2 == 0
    p = jnp.where(even, _SINC[4], _COSC[4])
    for t in range(3, -1, -1):
        p = p * r2 + jnp.where(even, _SINC[t], _COSC[t])
    pos = sign * p * jnp.where(even, rr, 1.0)
    out_ref[:, 0:D] = tok_ref[...] + pos + bias_ref[...]
    out_ref[:, D:W] = jnp.broadcast_to(ac_ref[...], (r, W - D))


def _build_ext_table(token_table, ac, bias2d):
    R = 2000
    return pl.pallas_call(
        _ext_table_body,
        grid=(V // R,),
        in_specs=[
            pl.BlockSpec((R, D), lambda i: (i, 0)),
            pl.BlockSpec((R, 1), lambda i: (i, 0)),
            pl.BlockSpec((1, D), lambda i: (0, 0)),
        ],
        out_specs=pl.BlockSpec((R, W), lambda i: (i, 0)),
        out_shape=jax.ShapeDtypeStruct((V, W), jnp.float32),
    )(token_table, ac, bias2d)


def _diff_body(ts_ref, out_ref):
    t = ts_ref[...]
    prev = jnp.concatenate([t[:, :1], t[:, :-1]], axis=1)
    out_ref[...] = t - prev


def _build_diff(ts2d):
    B, L = ts2d.shape
    R = 512
    return pl.pallas_call(
        _diff_body,
        grid=(B // R,),
        in_specs=[pl.BlockSpec((R, L), lambda i: (i, 0))],
        out_specs=pl.BlockSpec((R, L), lambda i: (i, 0)),
        out_shape=jax.ShapeDtypeStruct((B, L), jnp.int32),
    )(ts2d)


def _sc_body(pk_hbm, ext_hbm, wts_hbm, out_hbm,
             in0, in1, in2, in3, r0, r1, o0, o1,
             w_v, in_sem, g_sem, out_sem):
    in_v = [in0, in1, in2, in3]
    rows_v = [r0, r1]
    out_v = [o0, o1]
    wid = lax.axis_index("s") * NC + lax.axis_index("c")
    n_rows = out_hbm.shape[0] * 2
    rows_per_w = n_rows // NW
    base = wid * rows_per_w
    nch = rows_per_w // CH

    pltpu.sync_copy(wts_hbm, w_v)
    wlt = [w_v[pl.ds(j * LN, LN)] for j in range(4)]
    w0 = [w_v[pl.ds(D + j * LN, LN)] for j in range(4)]
    w1 = [w_v[pl.ds(2 * D + j * LN, LN)] for j in range(4)]

    def start_in(c, b):
        rb = base + jnp.minimum(c, nch - 1) * CH
        pltpu.async_copy(pk_hbm.at[:, pl.ds(rb, CH)], in_v[b], in_sem)

    def wait_in(b):
        pltpu.make_async_copy(pk_hbm.at[:, pl.ds(base, CH)], in_v[b],
                              in_sem).wait()

    def start_gather(bi, br):
        pltpu.async_copy(ext_hbm.at[in_v[bi].at[0]], rows_v[br], g_sem)

    def wait_gather(bi, br):
        pltpu.make_async_copy(ext_hbm.at[in_v[bi].at[0]], rows_v[br],
                              g_sem).wait()

    def start_out(c, br):
        rb2 = pl.multiple_of((base + c * CH) // 2, CH // 2)
        pltpu.async_copy(out_v[br], out_hbm.at[pl.ds(rb2, CH // 2), :],
                         out_sem)

    def drain_out(br):
        rb2 = pl.multiple_of(base // 2, CH // 2)
        pltpu.make_async_copy(out_v[br],
                              out_hbm.at[pl.ds(rb2, CH // 2), :],
                              out_sem).wait()

    def compute(b, br):
        inb, rowsb, outb = in_v[b], rows_v[br], out_v[br]

        def group_body(g, tk):
            gb = g * LN
            dv = inb[1, pl.ds(gb, LN)].astype(jnp.float32)
            f0g = inb[2, pl.ds(gb, LN)].astype(jnp.float32)
            f1g = inb[3, pl.ds(gb, LN)].astype(jnp.float32)
            for rr in range(0, LN, 2):
                rows = []
                for r in (rr, rr + 1):
                    i = gb + r
                    rsel = jnp.full((LN,), r, jnp.int32)
                    sb = rowsb[i, pl.ds(D, LN)] * jnp.take(dv, rsel)
                    rows.append((i, sb, jnp.take(f0g, rsel),
                                 jnp.take(f1g, rsel)))
                for j in range(4):
                    for i, sb, f0b, f1b in rows:
                        outb[i // 2, pl.ds((i % 2) * D + j * LN, LN)] = (
                            (rowsb[i, pl.ds(j * LN, LN)] + sb * wlt[j])
                            + (f0b * w0[j] + f1b * w1[j]))
            return tk

        lax.fori_loop(0, CH // LN, group_body, 0)

    def step(c, b, drain):
        wait_in((b + 1) % NBUF)
        start_gather((b + 1) % NBUF, (b + 1) % 2)
        start_in(c + 2, (b + 2) % NBUF)
        wait_gather((b + 1) % NBUF, b % 2)
        if drain:
            drain_out(b % 2)
        compute(b, b % 2)
        start_out(c, b % 2)

    start_in(0, 0)
    wait_in(0)
    start_gather(0, 0)
    start_in(1, 1)
    for b in range(NBUF):
        step(b, b, b >= 2)

    def outer(p, tk):
        for b in range(NBUF):
            step(p * NBUF + b, b, True)
        return tk

    lax.fori_loop(1, nch // NBUF, outer, 0)

    wait_in(1)
    wait_gather(0, 0)
    for b in range(2):
        drain_out(b)


def _run_sc(pk, ext, wts, n_rows):
    mesh = plsc.VectorSubcoreMesh(core_axis_name="c", subcore_axis_name="s")
    f = pl.kernel(
        _sc_body,
        out_type=jax.ShapeDtypeStruct((n_rows // 2, 2 * D), jnp.float32),
        mesh=mesh,
        compiler_params=pltpu.CompilerParams(use_tc_tiling_on_sc=True),
        scratch_types=(
            [pltpu.VMEM((4, CH), jnp.int32) for _ in range(NBUF)]
            + [pltpu.VMEM((CH, W), jnp.float32) for _ in range(2)]
            + [pltpu.VMEM((CH // 2, 2 * D), jnp.float32) for _ in range(2)]
            + [pltpu.VMEM((3 * D,), jnp.float32),
               pltpu.SemaphoreType.DMA,
               pltpu.SemaphoreType.DMA,
               pltpu.SemaphoreType.DMA]
        ),
    )
    return f(pk, ext, wts)


def kernel(x, token_table, ac, lt_w, lt_b, df_w, df_b):
    B, L, _ = x.shape
    n = B * L
    acts = x[:, :, 0].reshape(n)
    f0r = x[:, :, 2].reshape(n)
    f1r = x[:, :, 3].reshape(n)
    bias2d = (lt_b + df_b)[None, :]
    wts = jnp.concatenate([lt_w[:, 0], df_w[:, 0], df_w[:, 1]])
    ext = _build_ext_table(token_table, ac, bias2d)
    dif = _build_diff(x[:, :, 1]).reshape(n)
    pk = jnp.stack([acts, dif, f0r, f1r])
    out = _run_sc(pk, ext, wts, n)
    return out.reshape(B, L, D)


# speedup vs baseline: 1.8459x; 1.8459x over previous
import math

import jax
import jax.numpy as jnp
from jax import lax
from jax.experimental import pallas as pl
from jax.experimental.pallas import tpu as pltpu
from jax.experimental.pallas import tpu_sc as plsc

V = 100000
D = 64
W = 128
NC, NS, LN = 2, 16, 16
NW = NC * NS
CH = 128
NBUF = 4


_PI_HI = 3.140625
_PI_LO = math.pi - 3.140625
_SINC = (1.0, -1 / 6, 1 / 120, -1 / 5040, 1 / 362880)
_COSC = (1.0, -1 / 2, 1 / 24, -1 / 720, 1 / 40320)


def _ext_table_body(tok_ref, ac_ref, bias_ref, out_ref):
    i = pl.program_id(0)
    r = tok_ref.shape[0]
    row = (lax.broadcasted_iota(jnp.int32, (r, D), 0) + i * r
           ).astype(jnp.float32)
    col = lax.broadcasted_iota(jnp.int32, (r, D), 1)
    k = ((col // 2) * 2).astype(jnp.float32)
    ang = row * jnp.exp(k * (-math.log(10000.0) / D))
    q = jnp.floor(ang * (1.0 / math.pi) + 0.5)
    rr = (ang - q * _PI_HI) - q * _PI_LO
    r2 = rr * rr
    qh = q * 0.5
    sign = 1.0 - 4.0 * (qh - jnp.floor(qh))
    even = col % 2 == 0
    p = jnp.where(even, _SINC[4], _COSC[4])
    for t in range(3, -1, -1):
        p = p * r2 + jnp.where(even, _SINC[t], _COSC[t])
    pos = sign * p * jnp.where(even, rr, 1.0)
    out_ref[:, 0:D] = tok_ref[...] + pos + bias_ref[...]
    out_ref[:, D:W] = jnp.broadcast_to(ac_ref[...], (r, W - D))


def _build_ext_table(token_table, ac, bias2d):
    R = 2000
    return pl.pallas_call(
        _ext_table_body,
        grid=(V // R,),
        in_specs=[
            pl.BlockSpec((R, D), lambda i: (i, 0)),
            pl.BlockSpec((R, 1), lambda i: (i, 0)),
            pl.BlockSpec((1, D), lambda i: (0, 0)),
        ],
        out_specs=pl.BlockSpec((R, W), lambda i: (i, 0)),
        out_shape=jax.ShapeDtypeStruct((V, W), jnp.float32),
    )(token_table, ac, bias2d)


def _diff_body(ts_ref, out_ref):
    t = ts_ref[...]
    prev = jnp.concatenate([t[:, :1], t[:, :-1]], axis=1)
    out_ref[...] = t - prev


def _build_diff(ts2d):
    B, L = ts2d.shape
    R = 512
    return pl.pallas_call(
        _diff_body,
        grid=(B // R,),
        in_specs=[pl.BlockSpec((R, L), lambda i: (i, 0))],
        out_specs=pl.BlockSpec((R, L), lambda i: (i, 0)),
        out_shape=jax.ShapeDtypeStruct((B, L), jnp.int32),
    )(ts2d)


def _sc_body(pk_hbm, ext_hbm, wts_hbm, out_hbm,
             in0, in1, in2, in3, r0, r1, o0, o1,
             w_v, in_sem, g_sem, out_sem):
    in_v = [in0, in1, in2, in3]
    rows_v = [r0, r1]
    out_v = [o0, o1]
    wid = lax.axis_index("s") * NC + lax.axis_index("c")
    n_rows = out_hbm.shape[0]
    rows_per_w = n_rows // NW
    base = wid * rows_per_w
    nch = rows_per_w // CH

    pltpu.sync_copy(wts_hbm, w_v)
    wlt = [w_v[pl.ds(j * LN, LN)] for j in range(4)]
    w0 = [w_v[pl.ds(D + j * LN, LN)] for j in range(4)]
    w1 = [w_v[pl.ds(2 * D + j * LN, LN)] for j in range(4)]

    def start_in(c, b):
        rb = base + jnp.minimum(c, nch - 1) * CH
        pltpu.async_copy(pk_hbm.at[:, pl.ds(rb, CH)], in_v[b], in_sem)

    def wait_in(b):
        pltpu.make_async_copy(pk_hbm.at[:, pl.ds(base, CH)], in_v[b],
                              in_sem).wait()

    def start_gather(bi, br):
        pltpu.async_copy(ext_hbm.at[in_v[bi].at[0]], rows_v[br], g_sem)

    def wait_gather(bi, br):
        pltpu.make_async_copy(ext_hbm.at[in_v[bi].at[0]], rows_v[br],
                              g_sem).wait()

    def start_out(c, br):
        pltpu.async_copy(out_v[br],
                         out_hbm.at[pl.ds(base + c * CH, CH), :], out_sem)

    def drain_out(br):
        pltpu.make_async_copy(out_v[br], out_hbm.at[pl.ds(base, CH), :],
                              out_sem).wait()

    def compute(b, br):
        inb, rowsb, outb = in_v[b], rows_v[br], out_v[br]

        def group_body(g, tk):
            gb = g * LN
            dv = inb[1, pl.ds(gb, LN)].astype(jnp.float32)
            f0g = inb[2, pl.ds(gb, LN)].astype(jnp.float32)
            f1g = inb[3, pl.ds(gb, LN)].astype(jnp.float32)
            for rr in range(0, LN, 2):
                rows = []
                for r in (rr, rr + 1):
                    i = gb + r
                    rsel = jnp.full((LN,), r, jnp.int32)
                    sb = rowsb[i, pl.ds(D, LN)] * jnp.take(dv, rsel)
                    rows.append((i, sb, jnp.take(f0g, rsel),
                                 jnp.take(f1g, rsel)))
                for j in range(4):
                    for i, sb, f0b, f1b in rows:
                        outb[i, pl.ds(j * LN, LN)] = (
                            (rowsb[i, pl.ds(j * LN, LN)] + sb * wlt[j])
                            + (f0b * w0[j] + f1b * w1[j]))
            return tk

        lax.fori_loop(0, CH // LN, group_body, 0)

    def step(c, b, drain):
        wait_in((b + 1) % NBUF)
        start_gather((b + 1) % NBUF, (b + 1) % 2)
        start_in(c + 2, (b + 2) % NBUF)
        wait_gather((b + 1) % NBUF, b % 2)
        if drain:
            drain_out(b % 2)
        compute(b, b % 2)
        start_out(c, b % 2)

    start_in(0, 0)
    wait_in(0)
    start_gather(0, 0)
    start_in(1, 1)
    for b in range(NBUF):
        step(b, b, b >= 2)

    def outer(p, tk):
        for b in range(NBUF):
            step(p * NBUF + b, b, True)
        return tk

    lax.fori_loop(1, nch // NBUF, outer, 0)

    wait_in(1)
    wait_gather(0, 0)
    for b in range(2):
        drain_out(b)


def _run_sc(pk, ext, wts, n_rows):
    mesh = plsc.VectorSubcoreMesh(core_axis_name="c", subcore_axis_name="s")
    f = pl.kernel(
        _sc_body,
        out_type=jax.ShapeDtypeStruct((n_rows, D), jnp.float32),
        mesh=mesh,
        compiler_params=pltpu.CompilerParams(use_tc_tiling_on_sc=True),
        scratch_types=(
            [pltpu.VMEM((4, CH), jnp.int32) for _ in range(NBUF)]
            + [pltpu.VMEM((CH, W), jnp.float32) for _ in range(2)]
            + [pltpu.VMEM((CH, D), jnp.float32) for _ in range(2)]
            + [pltpu.VMEM((3 * D,), jnp.float32),
               pltpu.SemaphoreType.DMA,
               pltpu.SemaphoreType.DMA,
               pltpu.SemaphoreType.DMA]
        ),
    )
    return f(pk, ext, wts)


def kernel(x, token_table, ac, lt_w, lt_b, df_w, df_b):
    B, L, _ = x.shape
    n = B * L
    acts = x[:, :, 0].reshape(n)
    f0r = x[:, :, 2].reshape(n)
    f1r = x[:, :, 3].reshape(n)
    bias2d = (lt_b + df_b)[None, :]
    wts = jnp.concatenate([lt_w[:, 0], df_w[:, 0], df_w[:, 1]])
    ext = _build_ext_table(token_table, ac, bias2d)
    dif = _build_diff(x[:, :, 1]).reshape(n)
    pk = jnp.stack([acts, dif, f0r, f1r])
    out = _run_sc(pk, ext, wts, n)
    return out.reshape(B, L, D)
